# Initial kernel scaffold; baseline (speedup 1.0000x reference)
import jax
import jax.numpy as jnp
from jax.experimental import pallas as pl


def kernel(input_nodes, input_edges, node_emb, graph_token_emb, W_l, b_l, W_r):
    raise NotImplementedError("write your pallas kernel here")


# speedup vs baseline: 56.5512x; 56.5512x over previous
import functools

import jax
import jax.numpy as jnp
from jax import lax
from jax.experimental import pallas as pl
from jax.experimental.pallas import tpu as pltpu
from jax.experimental.pallas import tpu_sc as plsc

G = 2
NV = 10001
E = 160000
H = 128
NPAD = 10240
ROWS = 16
FLAT = ROWS * NPAD
NC = 2
NS = 16
EPAD = 163840
EPT = EPAD // NS
CH = 2048
NCH = EPT // CH
NPT = NPAD // NS
FPT = FLAT // NS


def _sc_body(ty_hbm, edges_hbm, out_hbm,
             ty_v, src_v, dst_v, ones_v, zeros_v, idx_v, idx_oh, shared):
    g = lax.axis_index("c")
    w = lax.axis_index("s")

    def fill_ones(i, _):
        ones_v[pl.ds(i * 16, 16)] = jnp.ones((16,), jnp.float32)
        return 0
    lax.fori_loop(0, CH // 16, fill_ones, 0)

    def fill_zeros(i, _):
        zeros_v[pl.ds(i * 16, 16)] = jnp.zeros((16,), jnp.float32)
        return 0
    lax.fori_loop(0, FPT // 16, fill_zeros, 0)

    pltpu.sync_copy(ty_hbm.at[g], ty_v)

    pltpu.sync_copy(zeros_v, shared.at[pl.ds(w * FPT, FPT)])
    plsc.subcore_barrier()

    def oh_vec(k, _):
        v0 = w * NPT + k * 16
        t16 = ty_v[pl.ds(v0, 16)]
        idx_oh[pl.ds(k * 16, 16)] = (t16 + 8) * NPAD + (v0 + lax.iota(jnp.int32, 16))
        return 0
    lax.fori_loop(0, NPT // 16, oh_vec, 0)
    pltpu.sync_copy(ones_v.at[pl.ds(0, NPT)], shared.at[idx_oh], add=True)

    def chunk(ch, _):
        base = w * EPT + ch * CH
        pltpu.sync_copy(edges_hbm.at[g, 0, pl.ds(base, CH)], src_v)
        pltpu.sync_copy(edges_hbm.at[g, 1, pl.ds(base, CH)], dst_v)

        def vec(k, _):
            s16 = src_v[pl.ds(k * 16, 16)]
            d16 = dst_v[pl.ds(k * 16, 16)]
            t16 = plsc.load_gather(ty_v, [s16])
            idx_v[pl.ds(k * 16, 16)] = t16 * NPAD + d16
            return 0
        lax.fori_loop(0, CH // 16, vec, 0)
        pltpu.sync_copy(ones_v, shared.at[idx_v], add=True)
        return 0
    lax.fori_loop(0, NCH, chunk, 0)

    plsc.subcore_barrier()

    pltpu.sync_copy(shared.at[pl.ds(w * FPT, FPT)],
                    out_hbm.at[g, pl.ds(w * FPT, FPT)])


@functools.cache
def _sc_hist():
    return functools.partial(
        pl.kernel,
        out_type=jax.ShapeDtypeStruct((G, FLAT), jnp.float32),
        mesh=plsc.VectorSubcoreMesh(core_axis_name="c", subcore_axis_name="s",
                                    num_cores=NC, num_subcores=NS),
        compiler_params=pltpu.CompilerParams(needs_layout_passes=False),
        scratch_types=[
            pltpu.VMEM((NPAD,), jnp.int32),
            pltpu.VMEM((CH,), jnp.int32),
            pltpu.VMEM((CH,), jnp.int32),
            pltpu.VMEM((CH,), jnp.float32),
            pltpu.VMEM((FPT,), jnp.float32),
            pltpu.VMEM((CH,), jnp.int32),
            pltpu.VMEM((NPT,), jnp.int32),
            pltpu.VMEM_SHARED((FLAT,), jnp.float32),
        ],
    )(_sc_body)


def _tc_body(scat_ref, t8_ref, wl_ref, wr_ref, bl_ref, gnf_ref, gef_ref):
    scat = scat_ref[0]
    cnt = jnp.sum(scat[0:8, :], axis=0, keepdims=True)
    recip = 1.0 / jnp.maximum(cnt, 1.0)
    rowid = lax.broadcasted_iota(jnp.int32, (ROWS, 1), 0)
    scaled = scat * jnp.where(rowid < 8, recip, 1.0)
    t8 = t8_ref[...]
    dn = (((1,), (1,)), ((), ()))
    twl = lax.dot_general(t8, wl_ref[...], dn, preferred_element_type=jnp.float32)
    twr = lax.dot_general(t8, wr_ref[...], dn, preferred_element_type=jnp.float32)
    wcat = jnp.concatenate([twl, twr], axis=0)
    d0 = (((0,), (0,)), ((), ()))
    gef_ref[0] = lax.dot_general(scaled, wcat, d0,
                                 preferred_element_type=jnp.float32) + bl_ref[...]
    gnf_ref[0] = lax.dot_general(scat[8:16, :], t8, d0,
                                 preferred_element_type=jnp.float32)


def _tc_dense(scat, t8, wl, wr, bl):
    NB = 1024
    grid = (G, NPAD // NB)
    return pl.pallas_call(
        _tc_body,
        grid=grid,
        in_specs=[
            pl.BlockSpec((1, ROWS, NB), lambda g, b: (g, 0, b)),
            pl.BlockSpec((8, H), lambda g, b: (0, 0)),
            pl.BlockSpec((H, H), lambda g, b: (0, 0)),
            pl.BlockSpec((H, H), lambda g, b: (0, 0)),
            pl.BlockSpec((1, H), lambda g, b: (0, 0)),
        ],
        out_specs=[
            pl.BlockSpec((1, NB, H), lambda g, b: (g, b, 0)),
            pl.BlockSpec((1, NB, H), lambda g, b: (g, b, 0)),
        ],
        out_shape=[
            jax.ShapeDtypeStruct((G, NV, H), jnp.float32),
            jax.ShapeDtypeStruct((G, NV, H), jnp.float32),
        ],
    )(scat, t8, wl, wr, bl)


@jax.jit
def kernel(input_nodes, input_edges, node_emb, graph_token_emb, W_l, b_l, W_r):
    input_nodes = input_nodes.astype(jnp.int32)
    edges = input_edges.astype(jnp.int32)

    ty = jnp.concatenate(
        [jnp.full((G, 1), 4, jnp.int32),
         input_nodes,
         jnp.zeros((G, NPAD - NV), jnp.int32)], axis=1)

    npad_e = EPAD - E
    r = jnp.arange(npad_e, dtype=jnp.int32)
    pad = jnp.stack([(r * 7919) % NV, NV + (r % (NPAD - NV - 1))], axis=0)
    edges_p = jnp.concatenate([edges, pad[None].repeat(G, axis=0)], axis=2)

    scat = _sc_hist()(ty, edges_p).reshape(G, ROWS, NPAD)

    t8 = jnp.concatenate(
        [node_emb, graph_token_emb, jnp.zeros((3, H), jnp.float32)], axis=0)
    gnf, gef = _tc_dense(scat, t8, W_l, W_r, b_l.reshape(1, H))
    return gnf, gef


# speedup vs baseline: 113.9368x; 2.0148x over previous
import functools

import jax
import jax.numpy as jnp
from jax import lax
from jax.experimental import pallas as pl
from jax.experimental.pallas import tpu as pltpu
from jax.experimental.pallas import tpu_sc as plsc

G = 2
NV = 10001
E = 160000
H = 128
NPAD = 10240
ROWS = 16
FLAT = ROWS * NPAD
NC = 2
NS = 16
EPAD = 163840
EPT = EPAD // NS
CH = 2048
NCH = EPT // CH
NPT = NPAD // NS
FPT = FLAT // NS


def _sc_body(ty_hbm, edges_hbm, out_hbm,
             ty_v, src_v, dst_v, ones_v, zeros_v, idx_v, idx_oh, shared):
    g = lax.axis_index("c")
    w = lax.axis_index("s")

    def fill_ones(i, _):
        ones_v[pl.ds(i * 16, 16)] = jnp.ones((16,), jnp.float32)
        return 0
    lax.fori_loop(0, CH // 16, fill_ones, 0)

    def fill_zeros(i, _):
        zeros_v[pl.ds(i * 16, 16)] = jnp.zeros((16,), jnp.float32)
        return 0
    lax.fori_loop(0, FPT // 16, fill_zeros, 0)

    pltpu.sync_copy(ty_hbm.at[g], ty_v)

    pltpu.sync_copy(zeros_v, shared.at[pl.ds(w * FPT, FPT)])
    plsc.subcore_barrier()

    def oh_vec(k, _):
        v0 = w * NPT + k * 16
        t16 = ty_v[pl.ds(v0, 16)]
        idx_oh[pl.ds(k * 16, 16)] = (t16 + 8) * NPAD + (v0 + lax.iota(jnp.int32, 16))
        return 0
    lax.fori_loop(0, NPT // 16, oh_vec, 0)
    pltpu.sync_copy(ones_v.at[pl.ds(0, NPT)], shared.at[idx_oh], add=True)

    def chunk(ch, _):
        base = w * EPT + ch * CH
        pltpu.sync_copy(edges_hbm.at[g, 0, pl.ds(base, CH)], src_v)
        pltpu.sync_copy(edges_hbm.at[g, 1, pl.ds(base, CH)], dst_v)

        def vec(k, _):
            s16 = src_v[pl.ds(k * 16, 16)]
            d16 = dst_v[pl.ds(k * 16, 16)]
            t16 = plsc.load_gather(ty_v, [s16])
            idx_v[pl.ds(k * 16, 16)] = t16 * NPAD + d16
            return 0
        lax.fori_loop(0, CH // 16, vec, 0)
        pltpu.sync_copy(ones_v, shared.at[idx_v], add=True)
        return 0
    lax.fori_loop(0, NCH, chunk, 0)

    plsc.subcore_barrier()

    pltpu.sync_copy(shared.at[pl.ds(w * FPT, FPT)],
                    out_hbm.at[g, pl.ds(w * FPT, FPT)])


@functools.cache
def _sc_hist():
    return functools.partial(
        pl.kernel,
        out_type=jax.ShapeDtypeStruct((G, FLAT), jnp.float32),
        mesh=plsc.VectorSubcoreMesh(core_axis_name="c", subcore_axis_name="s",
                                    num_cores=NC, num_subcores=NS),
        compiler_params=pltpu.CompilerParams(needs_layout_passes=False),
        scratch_types=[
            pltpu.VMEM((NPAD,), jnp.int32),
            pltpu.VMEM((CH,), jnp.int32),
            pltpu.VMEM((CH,), jnp.int32),
            pltpu.VMEM((CH,), jnp.float32),
            pltpu.VMEM((FPT,), jnp.float32),
            pltpu.VMEM((CH,), jnp.int32),
            pltpu.VMEM((NPT,), jnp.int32),
            pltpu.VMEM_SHARED((FLAT,), jnp.float32),
        ],
    )(_sc_body)


def _tc_body(scat_ref, t8_ref, wl_ref, wr_ref, bl_ref, gnf_ref, gef_ref):
    t8 = t8_ref[...]
    dn = (((1,), (1,)), ((), ()))
    twl = lax.dot_general(t8, wl_ref[...], dn, preferred_element_type=jnp.float32)
    twr = lax.dot_general(t8, wr_ref[...], dn, preferred_element_type=jnp.float32)
    wcat = jnp.concatenate([twl, twr], axis=0)
    d0 = (((0,), (0,)), ((), ()))
    rowid = lax.broadcasted_iota(jnp.int32, (ROWS, 1), 0)
    for g in range(G):
        scat = scat_ref[g]
        cnt = jnp.sum(scat[0:8, :], axis=0, keepdims=True)
        recip = 1.0 / jnp.maximum(cnt, 1.0)
        scaled = scat * jnp.where(rowid < 8, recip, 1.0)
        gef_ref[:, g, :] = lax.dot_general(
            scaled, wcat, d0, preferred_element_type=jnp.float32) + bl_ref[...]
        gnf_ref[:, g, :] = lax.dot_general(
            scat[8:16, :], t8, d0, preferred_element_type=jnp.float32)


def _tc_dense(scat, t8, wl, wr, bl):
    NB = 1024
    grid = (NPAD // NB,)
    return pl.pallas_call(
        _tc_body,
        grid=grid,
        in_specs=[
            pl.BlockSpec((G, ROWS, NB), lambda b: (0, 0, b)),
            pl.BlockSpec((8, H), lambda b: (0, 0)),
            pl.BlockSpec((H, H), lambda b: (0, 0)),
            pl.BlockSpec((H, H), lambda b: (0, 0)),
            pl.BlockSpec((1, H), lambda b: (0, 0)),
        ],
        out_specs=[
            pl.BlockSpec((NB, G, H), lambda b: (b, 0, 0)),
            pl.BlockSpec((NB, G, H), lambda b: (b, 0, 0)),
        ],
        out_shape=[
            jax.ShapeDtypeStruct((NV, G, H), jnp.float32),
            jax.ShapeDtypeStruct((NV, G, H), jnp.float32),
        ],
    )(scat, t8, wl, wr, bl)


@jax.jit
def kernel(input_nodes, input_edges, node_emb, graph_token_emb, W_l, b_l, W_r):
    input_nodes = input_nodes.astype(jnp.int32)
    edges = input_edges.astype(jnp.int32)

    ty = jnp.concatenate(
        [jnp.full((G, 1), 4, jnp.int32),
         input_nodes,
         jnp.zeros((G, NPAD - NV), jnp.int32)], axis=1)

    npad_e = EPAD - E
    r = jnp.arange(npad_e, dtype=jnp.int32)
    pad = jnp.stack([(r * 7919) % NV, NV + (r % (NPAD - NV - 1))], axis=0)
    edges_p = jnp.concatenate([edges, pad[None].repeat(G, axis=0)], axis=2)

    scat = _sc_hist()(ty, edges_p).reshape(G, ROWS, NPAD)

    t8 = jnp.concatenate(
        [node_emb, graph_token_emb, jnp.zeros((3, H), jnp.float32)], axis=0)
    gnf, gef = _tc_dense(scat, t8, W_l, W_r, b_l.reshape(1, H))
    return jnp.swapaxes(gnf, 0, 1), jnp.swapaxes(gef, 0, 1)


# speedup vs baseline: 122.7632x; 1.0775x over previous
import functools

import jax
import jax.numpy as jnp
from jax import lax
from jax.experimental import pallas as pl
from jax.experimental.pallas import tpu as pltpu
from jax.experimental.pallas import tpu_sc as plsc

G = 2
NV = 10001
E = 160000
H = 128
NPAD = 10240
ROWS = 16
FLAT = ROWS * NPAD
NC = 2
NS = 16
EPAD = 163840
EPT = EPAD // NS
CH = 2048
NCH = EPT // CH
NPT = NPAD // NS
FPT = FLAT // NS


def _sc_body(ty_hbm, edges_hbm, out_hbm,
             ty_v, src_v, dst_v, ones_v, zeros_v, idx_v, idx_oh, shared):
    g = lax.axis_index("c")
    w = lax.axis_index("s")

    def fill_ones(i, _):
        ones_v[pl.ds(i * 16, 16)] = jnp.ones((16,), jnp.float32)
        return 0
    lax.fori_loop(0, CH // 16, fill_ones, 0)

    def fill_zeros(i, _):
        zeros_v[pl.ds(i * 16, 16)] = jnp.zeros((16,), jnp.float32)
        return 0
    lax.fori_loop(0, FPT // 16, fill_zeros, 0)

    pltpu.sync_copy(ty_hbm.at[g], ty_v)

    pltpu.sync_copy(zeros_v, shared.at[pl.ds(w * FPT, FPT)])
    plsc.subcore_barrier()

    def oh_vec(k, _):
        v0 = w * NPT + k * 16
        t16 = ty_v[pl.ds(v0, 16)]
        idx_oh[pl.ds(k * 16, 16)] = (t16 + 8) * NPAD + (v0 + lax.iota(jnp.int32, 16))
        return 0
    lax.fori_loop(0, NPT // 16, oh_vec, 0)
    pltpu.sync_copy(ones_v.at[pl.ds(0, NPT)], shared.at[idx_oh], add=True)

    def chunk(ch, _):
        base = w * EPT + ch * CH
        pltpu.sync_copy(edges_hbm.at[g, 0, pl.ds(base, CH)], src_v)
        pltpu.sync_copy(edges_hbm.at[g, 1, pl.ds(base, CH)], dst_v)

        def vec(k, _):
            s16 = src_v[pl.ds(k * 16, 16)]
            d16 = dst_v[pl.ds(k * 16, 16)]
            t16 = plsc.load_gather(ty_v, [s16])
            idx_v[pl.ds(k * 16, 16)] = t16 * NPAD + d16
            return 0
        lax.fori_loop(0, CH // 16, vec, 0)
        pltpu.sync_copy(ones_v, shared.at[idx_v], add=True)
        return 0
    lax.fori_loop(0, NCH, chunk, 0)

    plsc.subcore_barrier()

    pltpu.sync_copy(shared.at[pl.ds(w * FPT, FPT)],
                    out_hbm.at[g, pl.ds(w * FPT, FPT)])


@functools.cache
def _sc_hist():
    return functools.partial(
        pl.kernel,
        out_type=jax.ShapeDtypeStruct((G, FLAT), jnp.float32),
        mesh=plsc.VectorSubcoreMesh(core_axis_name="c", subcore_axis_name="s",
                                    num_cores=NC, num_subcores=NS),
        compiler_params=pltpu.CompilerParams(needs_layout_passes=False),
        scratch_types=[
            pltpu.VMEM((NPAD,), jnp.int32),
            pltpu.VMEM((CH,), jnp.int32),
            pltpu.VMEM((CH,), jnp.int32),
            pltpu.VMEM((CH,), jnp.float32),
            pltpu.VMEM((FPT,), jnp.float32),
            pltpu.VMEM((CH,), jnp.int32),
            pltpu.VMEM((NPT,), jnp.int32),
            pltpu.VMEM_SHARED((FLAT,), jnp.float32),
        ],
    )(_sc_body)


NB = 1024
NBLK = NPAD // NB
NV_LAST = NV - (NBLK - 1) * NB


def _tc_body(scat_ref, t8_ref, wl_ref, wr_ref, bl_ref, gnf_hbm, gef_hbm,
             buf, buf2, sem):
    b = pl.program_id(0)
    slot = lax.rem(b, 2)
    t8 = t8_ref[...]
    dn = (((1,), (1,)), ((), ()))
    twl = lax.dot_general(t8, wl_ref[...], dn, preferred_element_type=jnp.float32)
    twr = lax.dot_general(t8, wr_ref[...], dn, preferred_element_type=jnp.float32)
    rhs = jnp.concatenate(
        [jnp.concatenate([twl, twr], axis=0),
         jnp.concatenate([jnp.zeros((8, H), jnp.float32), t8], axis=0)], axis=1)
    rowid = lax.broadcasted_iota(jnp.int32, (ROWS, 1), 0)
    d0 = (((0,), (0,)), ((), ()))

    def dmas(slot_, blk, rows):
        out = []
        for g in range(G):
            src = buf.at[slot_, g, pl.ds(0, rows), :]
            out.append(pltpu.make_async_copy(
                src.at[:, pl.ds(0, H)],
                gef_hbm.at[pl.ds(blk * NB, rows), g, :], sem.at[slot_]))
            out.append(pltpu.make_async_copy(
                src.at[:, pl.ds(H, H)],
                gnf_hbm.at[pl.ds(blk * NB, rows), g, :], sem.at[slot_]))
        return out

    @pl.when(b >= 2)
    def _():
        for c in dmas(slot, b - 2, NB):
            c.wait()

    bias = jnp.concatenate([bl_ref[...], jnp.zeros((1, H), jnp.float32)], axis=1)
    for g in range(G):
        scat = scat_ref[g]
        cnt = jnp.sum(scat[0:8, :], axis=0, keepdims=True)
        recip = 1.0 / jnp.maximum(cnt, 1.0)
        scaled = scat * jnp.where(rowid < 8, recip, 1.0)
        res = lax.dot_general(
            scaled, rhs, d0, preferred_element_type=jnp.float32) + bias
        buf[slot, g] = res
        buf2[g] = res[NV_LAST - 1:NV_LAST, :]

    @pl.when(b < NBLK - 1)
    def _():
        for c in dmas(slot, b, NB):
            c.start()

    @pl.when(b == NBLK - 1)
    def _():
        vlast = b * NB + NV_LAST - 1
        tail = dmas(slot, b, NV_LAST - 1)
        for g in range(G):
            tail.append(pltpu.make_async_copy(
                buf2.at[g, :, pl.ds(0, H)],
                gef_hbm.at[pl.ds(vlast, 1), g, :], sem.at[slot]))
            tail.append(pltpu.make_async_copy(
                buf2.at[g, :, pl.ds(H, H)],
                gnf_hbm.at[pl.ds(vlast, 1), g, :], sem.at[slot]))
        for c in tail:
            c.start()
        for c in dmas(1 - slot, b - 1, NB):
            c.wait()
        for c in tail:
            c.wait()


def _tc_dense(scat, t8, wl, wr, bl):
    return pl.pallas_call(
        _tc_body,
        grid=(NBLK,),
        in_specs=[
            pl.BlockSpec((G, ROWS, NB), lambda b: (0, 0, b)),
            pl.BlockSpec((8, H), lambda b: (0, 0)),
            pl.BlockSpec((H, H), lambda b: (0, 0)),
            pl.BlockSpec((H, H), lambda b: (0, 0)),
            pl.BlockSpec((1, H), lambda b: (0, 0)),
        ],
        out_specs=[
            pl.BlockSpec(memory_space=pl.ANY),
            pl.BlockSpec(memory_space=pl.ANY),
        ],
        out_shape=[
            jax.ShapeDtypeStruct((NV, G, H), jnp.float32),
            jax.ShapeDtypeStruct((NV, G, H), jnp.float32),
        ],
        scratch_shapes=[
            pltpu.VMEM((2, G, NB, 2 * H), jnp.float32),
            pltpu.VMEM((G, 1, 2 * H), jnp.float32),
            pltpu.SemaphoreType.DMA((2,)),
        ],
    )(scat, t8, wl, wr, bl)


@jax.jit
def kernel(input_nodes, input_edges, node_emb, graph_token_emb, W_l, b_l, W_r):
    input_nodes = input_nodes.astype(jnp.int32)
    edges = input_edges.astype(jnp.int32)

    ty = jnp.concatenate(
        [jnp.full((G, 1), 4, jnp.int32),
         input_nodes,
         jnp.zeros((G, NPAD - NV), jnp.int32)], axis=1)

    npad_e = EPAD - E
    r = jnp.arange(npad_e, dtype=jnp.int32)
    pad = jnp.stack([(r * 7919) % NV, NV + (r % (NPAD - NV - 1))], axis=0)
    edges_p = jnp.concatenate([edges, pad[None].repeat(G, axis=0)], axis=2)

    scat = _sc_hist()(ty, edges_p).reshape(G, ROWS, NPAD)

    t8 = jnp.concatenate(
        [node_emb, graph_token_emb, jnp.zeros((3, H), jnp.float32)], axis=0)
    gnf, gef = _tc_dense(scat, t8, W_l, W_r, b_l.reshape(1, H))
    return jnp.swapaxes(gnf, 0, 1), jnp.swapaxes(gef, 0, 1)


# speedup vs baseline: 150.3917x; 1.2251x over previous
import functools

import jax
import jax.numpy as jnp
from jax import lax
from jax.experimental import pallas as pl
from jax.experimental.pallas import tpu as pltpu
from jax.experimental.pallas import tpu_sc as plsc

G = 2
NV = 10001
E = 160000
H = 128
NPAD = 10240
ROWS = 16
FLAT = ROWS * NPAD
NC = 2
NS = 16
EPAD = 163840
EPT = EPAD // NS
CH = 2048
NCH = EPT // CH
NPT = NPAD // NS
FPT = FLAT // NS


def _sc_body(ty_hbm, edges_hbm, out_hbm,
             ty_v, src_a, src_b, dst_a, dst_b, ones_v, zeros_v,
             ix0, ix1, ix2, ix3, ix4, idx_oh, shared,
             sem_ty, sem_e, sem_s):
    src_v = (src_a, src_b)
    dst_v = (dst_a, dst_b)
    idx_v = (ix0, ix1, ix2, ix3, ix4)
    g = lax.axis_index("c")
    w = lax.axis_index("s")

    ty_dma = pltpu.async_copy(ty_hbm.at[g], ty_v, sem_ty)

    def fill_ones(i, _):
        ones_v[pl.ds(i * 16, 16)] = jnp.ones((16,), jnp.float32)
        return 0
    lax.fori_loop(0, CH // 16, fill_ones, 0)

    def fill_zeros(i, _):
        zeros_v[pl.ds(i * 16, 16)] = jnp.zeros((16,), jnp.float32)
        return 0
    lax.fori_loop(0, FPT // 16, fill_zeros, 0)

    pltpu.sync_copy(zeros_v, shared.at[pl.ds(w * FPT, FPT)])

    e_dmas = {}

    def fetch(c):
        base = w * EPT + c * CH
        b = c % 2
        e_dmas[c] = (
            pltpu.async_copy(edges_hbm.at[g, 0, pl.ds(base, CH)],
                             src_v[b], sem_e),
            pltpu.async_copy(edges_hbm.at[g, 1, pl.ds(base, CH)],
                             dst_v[b], sem_e),
        )

    fetch(0)
    ty_dma.wait()
    plsc.subcore_barrier()

    def oh_vec(k, _):
        v0 = w * NPT + k * 16
        t16 = ty_v[pl.ds(v0, 16)]
        idx_oh[pl.ds(k * 16, 16)] = (t16 + 8) * NPAD + (v0 + lax.iota(jnp.int32, 16))
        return 0
    lax.fori_loop(0, NPT // 16, oh_vec, 0)
    scats = [pltpu.async_copy(ones_v.at[pl.ds(0, NPT)], shared.at[idx_oh],
                              sem_s, add=True)]

    for c in range(NCH):
        b = c % 2
        if c + 1 < NCH:
            fetch(c + 1)
        for d in e_dmas.pop(c):
            d.wait()
        sv, dv, iv = src_v[b], dst_v[b], idx_v[c]

        def vec(k, _, sv=sv, dv=dv, iv=iv):
            s16 = sv[pl.ds(k * 16, 16)]
            d16 = dv[pl.ds(k * 16, 16)]
            t16 = plsc.load_gather(ty_v, [s16])
            iv[pl.ds(k * 16, 16)] = t16 * NPAD + d16
            return 0
        lax.fori_loop(0, CH // 16, vec, 0)
        scats.append(pltpu.async_copy(ones_v, shared.at[iv], sem_s, add=True))

    for s in scats:
        s.wait()
    plsc.subcore_barrier()

    pltpu.sync_copy(shared.at[pl.ds(w * FPT, FPT)],
                    out_hbm.at[g, pl.ds(w * FPT, FPT)])


@functools.cache
def _sc_hist():
    return functools.partial(
        pl.kernel,
        out_type=jax.ShapeDtypeStruct((G, FLAT), jnp.float32),
        mesh=plsc.VectorSubcoreMesh(core_axis_name="c", subcore_axis_name="s",
                                    num_cores=NC, num_subcores=NS),
        compiler_params=pltpu.CompilerParams(needs_layout_passes=False),
        scratch_types=[
            pltpu.VMEM((NPAD,), jnp.int32),
            pltpu.VMEM((CH,), jnp.int32),
            pltpu.VMEM((CH,), jnp.int32),
            pltpu.VMEM((CH,), jnp.int32),
            pltpu.VMEM((CH,), jnp.int32),
            pltpu.VMEM((CH,), jnp.float32),
            pltpu.VMEM((FPT,), jnp.float32),
            pltpu.VMEM((CH,), jnp.int32),
            pltpu.VMEM((CH,), jnp.int32),
            pltpu.VMEM((CH,), jnp.int32),
            pltpu.VMEM((CH,), jnp.int32),
            pltpu.VMEM((CH,), jnp.int32),
            pltpu.VMEM((NPT,), jnp.int32),
            pltpu.VMEM_SHARED((FLAT,), jnp.float32),
            pltpu.SemaphoreType.DMA,
            pltpu.SemaphoreType.DMA,
            pltpu.SemaphoreType.DMA,
        ],
    )(_sc_body)


NB = 1024
NBLK = NPAD // NB
NV_LAST = NV - (NBLK - 1) * NB


def _tc_body(scat_ref, t8_ref, wl_ref, wr_ref, bl_ref, gnf_hbm, gef_hbm,
             buf, buf2, sem):
    b = pl.program_id(0)
    slot = lax.rem(b, 2)
    t8 = t8_ref[...]
    dn = (((1,), (1,)), ((), ()))
    twl = lax.dot_general(t8, wl_ref[...], dn, preferred_element_type=jnp.float32)
    twr = lax.dot_general(t8, wr_ref[...], dn, preferred_element_type=jnp.float32)
    rhs = jnp.concatenate(
        [jnp.concatenate([twl, twr], axis=0),
         jnp.concatenate([jnp.zeros((8, H), jnp.float32), t8], axis=0)], axis=1)
    rowid = lax.broadcasted_iota(jnp.int32, (ROWS, 1), 0)
    d0 = (((0,), (0,)), ((), ()))

    def dmas(slot_, blk, rows):
        out = []
        for g in range(G):
            src = buf.at[slot_, g, pl.ds(0, rows), :]
            out.append(pltpu.make_async_copy(
                src.at[:, pl.ds(0, H)],
                gef_hbm.at[pl.ds(blk * NB, rows), g, :], sem.at[slot_]))
            out.append(pltpu.make_async_copy(
                src.at[:, pl.ds(H, H)],
                gnf_hbm.at[pl.ds(blk * NB, rows), g, :], sem.at[slot_]))
        return out

    @pl.when(b >= 2)
    def _():
        for c in dmas(slot, b - 2, NB):
            c.wait()

    bias = jnp.concatenate([bl_ref[...], jnp.zeros((1, H), jnp.float32)], axis=1)
    for g in range(G):
        scat = scat_ref[g]
        cnt = jnp.sum(scat[0:8, :], axis=0, keepdims=True)
        recip = 1.0 / jnp.maximum(cnt, 1.0)
        scaled = scat * jnp.where(rowid < 8, recip, 1.0)
        res = lax.dot_general(
            scaled, rhs, d0, preferred_element_type=jnp.float32) + bias
        buf[slot, g] = res
        buf2[g] = res[NV_LAST - 1:NV_LAST, :]

    @pl.when(b < NBLK - 1)
    def _():
        for c in dmas(slot, b, NB):
            c.start()

    @pl.when(b == NBLK - 1)
    def _():
        vlast = b * NB + NV_LAST - 1
        tail = dmas(slot, b, NV_LAST - 1)
        for g in range(G):
            tail.append(pltpu.make_async_copy(
                buf2.at[g, :, pl.ds(0, H)],
                gef_hbm.at[pl.ds(vlast, 1), g, :], sem.at[slot]))
            tail.append(pltpu.make_async_copy(
                buf2.at[g, :, pl.ds(H, H)],
                gnf_hbm.at[pl.ds(vlast, 1), g, :], sem.at[slot]))
        for c in tail:
            c.start()
        for c in dmas(1 - slot, b - 1, NB):
            c.wait()
        for c in tail:
            c.wait()


def _tc_dense(scat, t8, wl, wr, bl):
    return pl.pallas_call(
        _tc_body,
        grid=(NBLK,),
        in_specs=[
            pl.BlockSpec((G, ROWS, NB), lambda b: (0, 0, b)),
            pl.BlockSpec((8, H), lambda b: (0, 0)),
            pl.BlockSpec((H, H), lambda b: (0, 0)),
            pl.BlockSpec((H, H), lambda b: (0, 0)),
            pl.BlockSpec((1, H), lambda b: (0, 0)),
        ],
        out_specs=[
            pl.BlockSpec(memory_space=pl.ANY),
            pl.BlockSpec(memory_space=pl.ANY),
        ],
        out_shape=[
            jax.ShapeDtypeStruct((NV, G, H), jnp.float32),
            jax.ShapeDtypeStruct((NV, G, H), jnp.float32),
        ],
        scratch_shapes=[
            pltpu.VMEM((2, G, NB, 2 * H), jnp.float32),
            pltpu.VMEM((G, 1, 2 * H), jnp.float32),
            pltpu.SemaphoreType.DMA((2,)),
        ],
    )(scat, t8, wl, wr, bl)


@jax.jit
def kernel(input_nodes, input_edges, node_emb, graph_token_emb, W_l, b_l, W_r):
    input_nodes = input_nodes.astype(jnp.int32)
    edges = input_edges.astype(jnp.int32)

    ty = jnp.concatenate(
        [jnp.full((G, 1), 4, jnp.int32),
         input_nodes,
         jnp.zeros((G, NPAD - NV), jnp.int32)], axis=1)

    npad_e = EPAD - E
    r = jnp.arange(npad_e, dtype=jnp.int32)
    pad = jnp.stack([(r * 7919) % NV, NV + (r % (NPAD - NV - 1))], axis=0)
    edges_p = jnp.concatenate([edges, pad[None].repeat(G, axis=0)], axis=2)

    scat = _sc_hist()(ty, edges_p).reshape(G, ROWS, NPAD)

    t8 = jnp.concatenate(
        [node_emb, graph_token_emb, jnp.zeros((3, H), jnp.float32)], axis=0)
    gnf, gef = _tc_dense(scat, t8, W_l, W_r, b_l.reshape(1, H))
    return jnp.swapaxes(gnf, 0, 1), jnp.swapaxes(gef, 0, 1)
